# baseline (speedup 1.0000x reference)
import functools

import jax
import jax.numpy as jnp
from jax import lax
from jax.experimental import pallas as pl
from jax.experimental.pallas import tpu as pltpu
from jax.experimental.pallas import tpu_sc as plsc

VOCAB = 1000000
MAX_LEN = 200
EMBED_DIM = 32
BATCH = 16384
SEQ = 200

NUM_CORES = 2
NUM_SUBCORES = 16
NUM_WORKERS = NUM_CORES * NUM_SUBCORES

SEQS_PER_WORKER = BATCH // NUM_WORKERS
SEQS_PER_CHUNK = 4
STEPS = SEQS_PER_WORKER // SEQS_PER_CHUNK
CHUNK = SEQS_PER_CHUNK * SEQ
NBUF = 4


def _sc_body(x_hbm, tok_hbm, pos_hbm, out_hbm, *scratch):
    idx_bufs = scratch[0:NBUF]
    row_bufs = scratch[NBUF:2 * NBUF]
    pos_c = scratch[2 * NBUF]
    sem_g = scratch[2 * NBUF + 1:2 * NBUF + 1 + NBUF]
    sem_s = scratch[2 * NBUF + 1 + NBUF:2 * NBUF + 1 + 2 * NBUF]
    sem_i = scratch[2 * NBUF + 1 + 2 * NBUF:2 * NBUF + 1 + 3 * NBUF]

    wid = lax.axis_index("s") * NUM_CORES + lax.axis_index("c")
    base = wid * SEQS_PER_WORKER

    pltpu.sync_copy(pos_hbm, pos_c)

    def copy_idx_in(b, bat0, copy):
        for k in range(SEQS_PER_CHUNK):
            copy(x_hbm.at[bat0 + k],
                 idx_bufs[b].at[pl.ds(k * SEQ, SEQ)], sem_i[b])

    for b in range(NBUF):
        copy_idx_in(b, base + b * SEQS_PER_CHUNK,
                    lambda s, d, m: pltpu.sync_copy(s, d))
        pltpu.async_copy(tok_hbm.at[idx_bufs[b]], row_bufs[b], sem_g[b])

    def outer(G, _):
        for b in range(NBUF):
            g = G * NBUF + b
            bat0 = base + g * SEQS_PER_CHUNK
            pltpu.make_async_copy(
                tok_hbm.at[idx_bufs[b]], row_bufs[b], sem_g[b]).wait()

            @pl.when(g + NBUF < STEPS)
            def _(b=b, g=g):
                copy_idx_in(b, base + (g + NBUF) * SEQS_PER_CHUNK,
                            pltpu.async_copy)

            @plsc.parallel_loop(0, SEQ, unroll=8)
            def _(j, b=b):
                for h in range(EMBED_DIM // 16):
                    p = pos_c[j, pl.ds(h * 16, 16)]
                    for k in range(SEQS_PER_CHUNK):
                        plsc.addupdate(
                            row_bufs[b].at[k * SEQ + j, pl.ds(h * 16, 16)], p)

            for k in range(SEQS_PER_CHUNK):
                pltpu.async_copy(
                    row_bufs[b].at[pl.ds(k * SEQ, SEQ)],
                    out_hbm.at[bat0 + k], sem_s[b])

            bp = (b - 1) % NBUF
            c = g - 1 + NBUF
            pred = (G >= 1) if b == 0 else (c < STEPS)

            @pl.when(pred)
            def _(bp=bp, c=c):
                for k in range(SEQS_PER_CHUNK):
                    pltpu.make_async_copy(
                        row_bufs[bp].at[pl.ds(k * SEQ, SEQ)],
                        out_hbm.at[k], sem_s[bp]).wait()
                    pltpu.make_async_copy(
                        x_hbm.at[k], idx_bufs[bp].at[pl.ds(k * SEQ, SEQ)],
                        sem_i[bp]).wait()
                pltpu.async_copy(
                    tok_hbm.at[idx_bufs[bp]], row_bufs[bp], sem_g[bp])
        return 0

    lax.fori_loop(0, STEPS // NBUF, outer, 0)

    for b in range(NBUF):
        for k in range(SEQS_PER_CHUNK):
            pltpu.make_async_copy(
                row_bufs[b].at[pl.ds(k * SEQ, SEQ)],
                out_hbm.at[k], sem_s[b]).wait()


def kernel(x, token_table, pos_table):
    mesh = plsc.VectorSubcoreMesh(core_axis_name="c", subcore_axis_name="s")
    scratch = (
        [pltpu.VMEM((CHUNK,), jnp.int32) for _ in range(NBUF)]
        + [pltpu.VMEM((CHUNK, EMBED_DIM), jnp.float32) for _ in range(NBUF)]
        + [pltpu.VMEM((SEQ, EMBED_DIM), jnp.float32)]
        + [pltpu.SemaphoreType.DMA for _ in range(3 * NBUF)]
    )
    k = functools.partial(
        pl.kernel,
        mesh=mesh,
        compiler_params=pltpu.CompilerParams(use_tc_tiling_on_sc=False),
        out_type=jax.ShapeDtypeStruct((BATCH, SEQ, EMBED_DIM), jnp.float32),
        scratch_types=scratch,
    )(_sc_body)
    return k(x.astype(jnp.int32), token_table, pos_table)
